# baseline (speedup 1.0000x reference)
import jax
import jax.numpy as jnp
from jax.experimental import pallas as pl
from jax.experimental.pallas import tpu as pltpu

_IN = 16
_OUT = 32
_BC = 16384


def _fused_body(minmax_ref, x_ref, p_ref, o_ref):
    wt = p_ref[:, 0:_IN]
    b = p_ref[:, _IN:_IN + 1]
    acc = jnp.dot(wt, x_ref[...], preferred_element_type=jnp.float32)
    acc = acc + b
    o_ref[...] = jnp.minimum(jnp.maximum(acc, minmax_ref[0]), minmax_ref[1])


@jax.jit
def kernel(x, w_packed, b_packed, min_value, max_value):
    B = x.shape[0]
    minmax = jnp.stack([jnp.asarray(min_value, jnp.float32).reshape(()),
                        jnp.asarray(max_value, jnp.float32).reshape(())])
    params = jnp.concatenate(
        [w_packed[:_IN, :_OUT].T, b_packed[0:1, :_OUT].T], axis=1)

    xt = x.T
    nc = pl.cdiv(B, _BC)
    out_t = pl.pallas_call(
        _fused_body,
        out_shape=jax.ShapeDtypeStruct((_OUT, B), jnp.float32),
        grid=(nc,),
        in_specs=[
            pl.BlockSpec(memory_space=pltpu.SMEM),
            pl.BlockSpec((_IN, _BC), lambda i: (0, i)),
            pl.BlockSpec((_OUT, _IN + 1), lambda i: (0, 0)),
        ],
        out_specs=pl.BlockSpec((_OUT, _BC), lambda i: (0, i)),
        compiler_params=pltpu.CompilerParams(
            dimension_semantics=("parallel",)),
    )(minmax, xt, params)
    return out_t.T


# speedup vs baseline: 1.3400x; 1.3400x over previous
import jax
import jax.numpy as jnp
from jax.experimental import pallas as pl
from jax.experimental.pallas import tpu as pltpu

_IN = 16
_OUT = 32
_BC = 65536


def _fused_body(minmax_ref, x_ref, p_ref, o_ref):
    wt = p_ref[:, 0:_IN]
    b = p_ref[:, _IN:_IN + 1]
    acc = jnp.dot(wt, x_ref[...], preferred_element_type=jnp.float32)
    acc = acc + b
    o_ref[...] = jnp.minimum(jnp.maximum(acc, minmax_ref[0]), minmax_ref[1])


@jax.jit
def kernel(x, w_packed, b_packed, min_value, max_value):
    B = x.shape[0]
    minmax = jnp.stack([jnp.asarray(min_value, jnp.float32).reshape(()),
                        jnp.asarray(max_value, jnp.float32).reshape(())])
    params = jnp.concatenate(
        [w_packed[:_IN, :_OUT].T, b_packed[0:1, :_OUT].T], axis=1)

    xt = x.T
    nc = pl.cdiv(B, _BC)
    out_t = pl.pallas_call(
        _fused_body,
        out_shape=jax.ShapeDtypeStruct((_OUT, B), jnp.float32),
        grid=(nc,),
        in_specs=[
            pl.BlockSpec(memory_space=pltpu.SMEM),
            pl.BlockSpec((_IN, _BC), lambda i: (0, i)),
            pl.BlockSpec((_OUT, _IN + 1), lambda i: (0, 0)),
        ],
        out_specs=pl.BlockSpec((_OUT, _BC), lambda i: (0, i)),
        compiler_params=pltpu.CompilerParams(
            dimension_semantics=("parallel",),
            vmem_limit_bytes=48 << 20),
    )(minmax, xt, params)
    return out_t.T


# speedup vs baseline: 1.4035x; 1.0474x over previous
import jax
import jax.numpy as jnp
from jax.experimental import pallas as pl
from jax.experimental.pallas import tpu as pltpu

_IN = 16
_OUT = 32
_BC = 131072


def _fused_body(minmax_ref, x_ref, p_ref, o_ref):
    wt = p_ref[:, 0:_IN]
    b = p_ref[:, _IN:_IN + 1]
    acc = jnp.dot(wt, x_ref[...], preferred_element_type=jnp.float32)
    acc = acc + b
    o_ref[...] = jnp.minimum(jnp.maximum(acc, minmax_ref[0]), minmax_ref[1])


@jax.jit
def kernel(x, w_packed, b_packed, min_value, max_value):
    B = x.shape[0]
    minmax = jnp.stack([jnp.asarray(min_value, jnp.float32).reshape(()),
                        jnp.asarray(max_value, jnp.float32).reshape(())])
    params = jnp.concatenate(
        [w_packed[:_IN, :_OUT].T, b_packed[0:1, :_OUT].T], axis=1)

    xt = x.T
    nc = pl.cdiv(B, _BC)
    out_t = pl.pallas_call(
        _fused_body,
        out_shape=jax.ShapeDtypeStruct((_OUT, B), jnp.float32),
        grid=(nc,),
        in_specs=[
            pl.BlockSpec(memory_space=pltpu.SMEM),
            pl.BlockSpec((_IN, _BC), lambda i: (0, i)),
            pl.BlockSpec((_OUT, _IN + 1), lambda i: (0, 0)),
        ],
        out_specs=pl.BlockSpec((_OUT, _BC), lambda i: (0, i)),
        compiler_params=pltpu.CompilerParams(
            dimension_semantics=("parallel",),
            vmem_limit_bytes=48 << 20),
    )(minmax, xt, params)
    return out_t.T


# speedup vs baseline: 1.4715x; 1.0485x over previous
import jax
import jax.numpy as jnp
from jax.experimental import pallas as pl
from jax.experimental.pallas import tpu as pltpu

_IN = 16
_OUT = 32
_BC = 131072


def _fused_body(lo_ref, hi_ref, x_ref, w_ref, b_ref, o_ref):
    wb = jnp.concatenate([w_ref[0:_IN, 0:_OUT], b_ref[0:1, 0:_OUT]], axis=0)
    ones = jnp.ones((1, x_ref.shape[1]), jnp.float32)
    xa = jnp.concatenate([x_ref[...], ones], axis=0)
    acc = jax.lax.dot_general(
        wb, xa, (((0,), (0,)), ((), ())),
        preferred_element_type=jnp.float32)
    o_ref[...] = jnp.minimum(jnp.maximum(acc, lo_ref[0]), hi_ref[0])


@jax.jit
def kernel(x, w_packed, b_packed, min_value, max_value):
    B = x.shape[0]
    lo = jnp.asarray(min_value, jnp.float32).reshape(1)
    hi = jnp.asarray(max_value, jnp.float32).reshape(1)
    xt = x.T
    nc = pl.cdiv(B, _BC)
    out_t = pl.pallas_call(
        _fused_body,
        out_shape=jax.ShapeDtypeStruct((_OUT, B), jnp.float32),
        grid=(nc,),
        in_specs=[
            pl.BlockSpec(memory_space=pltpu.SMEM),
            pl.BlockSpec(memory_space=pltpu.SMEM),
            pl.BlockSpec((_IN, _BC), lambda i: (0, i)),
            pl.BlockSpec((128, 256), lambda i: (0, 0)),
            pl.BlockSpec((1, 256), lambda i: (0, 0)),
        ],
        out_specs=pl.BlockSpec((_OUT, _BC), lambda i: (0, i)),
        compiler_params=pltpu.CompilerParams(
            dimension_semantics=("parallel",),
            vmem_limit_bytes=56 << 20),
    )(lo, hi, xt, w_packed, b_packed)
    return out_t.T


# speedup vs baseline: 1.4805x; 1.0061x over previous
import jax
import jax.numpy as jnp
from jax.experimental import pallas as pl
from jax.experimental.pallas import tpu as pltpu

_IN = 16
_OUT = 32
_CH = 32768
_NBUF = 4


def _body(lo_ref, hi_ref, x_hbm, w_ref, b_ref, o_hbm, xbuf, obuf,
          in_sem, out_sem, *, cols_per_core, n_steps):
    core = pl.program_id(0)
    base = core * cols_per_core

    wb = jnp.concatenate([w_ref[0:_IN, 0:_OUT], b_ref[0:1, 0:_OUT]], axis=0)
    lo = lo_ref[0]
    hi = hi_ref[0]

    def start_in(i, slot):
        pltpu.make_async_copy(
            x_hbm.at[:, pl.ds(base + i * _CH, _CH)],
            xbuf.at[slot], in_sem.at[slot]).start()

    def wait_in(slot):
        pltpu.make_async_copy(
            x_hbm.at[:, pl.ds(0, _CH)], xbuf.at[slot],
            in_sem.at[slot]).wait()

    def start_out(i, slot):
        pltpu.make_async_copy(
            obuf.at[slot], o_hbm.at[:, pl.ds(base + i * _CH, _CH)],
            out_sem.at[slot]).start()

    def wait_out(slot):
        pltpu.make_async_copy(
            obuf.at[slot], o_hbm.at[:, pl.ds(0, _CH)],
            out_sem.at[slot]).wait()

    for i in range(min(_NBUF, n_steps)):
        start_in(i, i)

    ones = jnp.ones((1, _CH), jnp.float32)
    for i in range(n_steps):
        slot = i % _NBUF
        wait_in(slot)
        if i >= _NBUF:
            wait_out(slot)
        xa = jnp.concatenate([xbuf[slot], ones], axis=0)
        acc = jax.lax.dot_general(
            wb, xa, (((0,), (0,)), ((), ())),
            preferred_element_type=jnp.float32)
        obuf[slot] = jnp.minimum(jnp.maximum(acc, lo), hi)
        start_out(i, slot)
        nxt = i + _NBUF
        if nxt < n_steps:
            start_in(nxt, slot)
    for i in range(max(0, n_steps - _NBUF), n_steps):
        wait_out(i % _NBUF)


@jax.jit
def kernel(x, w_packed, b_packed, min_value, max_value):
    import functools
    B = x.shape[0]
    cols_per_core = B // 2
    n_steps = cols_per_core // _CH
    lo = jnp.asarray(min_value, jnp.float32).reshape(1)
    hi = jnp.asarray(max_value, jnp.float32).reshape(1)
    xt = x.T
    out_t = pl.pallas_call(
        functools.partial(_body, cols_per_core=cols_per_core,
                          n_steps=n_steps),
        out_shape=jax.ShapeDtypeStruct((_OUT, B), jnp.float32),
        grid=(2,),
        in_specs=[
            pl.BlockSpec(memory_space=pltpu.SMEM),
            pl.BlockSpec(memory_space=pltpu.SMEM),
            pl.BlockSpec(memory_space=pl.ANY),
            pl.BlockSpec(memory_space=pltpu.VMEM),
            pl.BlockSpec(memory_space=pltpu.VMEM),
        ],
        out_specs=pl.BlockSpec(memory_space=pl.ANY),
        scratch_shapes=[
            pltpu.VMEM((_NBUF, _IN, _CH), jnp.float32),
            pltpu.VMEM((_NBUF, _OUT, _CH), jnp.float32),
            pltpu.SemaphoreType.DMA((_NBUF,)),
            pltpu.SemaphoreType.DMA((_NBUF,)),
        ],
        compiler_params=pltpu.CompilerParams(
            dimension_semantics=("parallel",),
            vmem_limit_bytes=56 << 20),
    )(lo, hi, xt, w_packed, b_packed)
    return out_t.T
